# baseline (speedup 1.0000x reference)
import jax
import jax.numpy as jnp
from jax.experimental import pallas as pl

_MAX_SEQ_LENGTH = 2048
_BLOCK_S = 512


def _pad_body(n_in_blocks, x_ref, o_ref):
    j = pl.program_id(1)

    @pl.when(j < n_in_blocks)
    def _copy():
        o_ref[...] = x_ref[...]

    @pl.when(j >= n_in_blocks)
    def _zero():
        o_ref[...] = jnp.zeros_like(o_ref)


def kernel(x):
    b, s, f = x.shape
    out_s = _MAX_SEQ_LENGTH
    blk = _BLOCK_S
    n_in_blocks = s // blk
    n_out_blocks = out_s // blk

    import functools
    body = functools.partial(_pad_body, n_in_blocks)

    return pl.pallas_call(
        body,
        grid=(b, n_out_blocks),
        in_specs=[
            pl.BlockSpec(
                (1, blk, f),
                lambda i, j: (i, jnp.minimum(j, n_in_blocks - 1), 0),
            )
        ],
        out_specs=pl.BlockSpec((1, blk, f), lambda i, j: (i, j, 0)),
        out_shape=jax.ShapeDtypeStruct((b, out_s, f), x.dtype),
    )(x)
